# Initial kernel scaffold; baseline (speedup 1.0000x reference)
import jax
import jax.numpy as jnp
from jax.experimental import pallas as pl


def kernel(x, We, be, Wn, bn, Wexp, bexp, noise_uniform):
    raise NotImplementedError("write your pallas kernel here")


# speedup vs baseline: 1.3548x; 1.3548x over previous
import jax
import jax.numpy as jnp
from jax.experimental import pallas as pl


def _moe_block(x_ref, u_ref, we_ref, wn_ref, wexp_ref, bexp_ref, o_ref):
    xb = x_ref[0]

    el = jnp.dot(we_ref[...], xb, preferred_element_type=jnp.float32)
    nl = jnp.dot(wn_ref[...], xb, preferred_element_type=jnp.float32)
    nl_max = jnp.max(nl, axis=0, keepdims=True)
    nl_exp = jnp.exp(nl - nl_max)
    noise = u_ref[0] * (nl_exp / jnp.sum(nl_exp, axis=0, keepdims=True))
    logits = el + noise

    lmin = jnp.min(logits, axis=0, keepdims=True)
    eidx = jax.lax.broadcasted_iota(jnp.int32, logits.shape, 0)
    drop = jnp.max(jnp.where(logits == lmin, eidx, -1), axis=0, keepdims=True)
    keep = eidx != drop
    lmax = jnp.max(logits, axis=0, keepdims=True)
    ex = jnp.where(keep, jnp.exp(logits - lmax), 0.0)
    gates = ex / jnp.sum(ex, axis=0, keepdims=True)
    s = jnp.sum(gates, axis=0, keepdims=True)

    y = jnp.dot(
        wexp_ref[...].astype(jnp.bfloat16),
        xb.astype(jnp.bfloat16),
        preferred_element_type=jnp.float32,
    )
    o_ref[0] = (y + bexp_ref[...]) * s


def kernel(x, We, be, Wn, bn, Wexp, bexp, noise_uniform):
    B, C, N, P = x.shape
    E = We.shape[0]
    O = Wexp.shape[0]
    NP = N * P
    T = 2048

    x3 = x.reshape(B, C, NP)
    u3 = jnp.transpose(noise_uniform.reshape(B, NP, E), (0, 2, 1))
    bexp2 = bexp.reshape(O, 1)

    out = pl.pallas_call(
        _moe_block,
        grid=(B, NP // T),
        in_specs=[
            pl.BlockSpec((1, C, T), lambda b, i: (b, 0, i)),
            pl.BlockSpec((1, E, T), lambda b, i: (b, 0, i)),
            pl.BlockSpec((E, C), lambda b, i: (0, 0)),
            pl.BlockSpec((E, C), lambda b, i: (0, 0)),
            pl.BlockSpec((O, C), lambda b, i: (0, 0)),
            pl.BlockSpec((O, 1), lambda b, i: (0, 0)),
        ],
        out_specs=pl.BlockSpec((1, O, T), lambda b, i: (b, 0, i)),
        out_shape=jax.ShapeDtypeStruct((B, O, NP), x.dtype),
    )(x3, u3, We, Wn, Wexp, bexp2)
    return out.reshape(B, O, N, P)


# speedup vs baseline: 1.3616x; 1.0050x over previous
import jax
import jax.numpy as jnp
from jax.experimental import pallas as pl


def _moe_block(x_ref, u_ref, wg_ref, wexp_ref, bexp_ref, o_ref):
    xb16 = x_ref[0].astype(jnp.bfloat16)
    E = u_ref.shape[1]

    g = jnp.dot(wg_ref[...].astype(jnp.bfloat16), xb16,
                preferred_element_type=jnp.float32)
    el = g[:E]
    nl = g[E:]
    nl_max = jnp.max(nl, axis=0, keepdims=True)
    nl_exp = jnp.exp(nl - nl_max)
    noise = u_ref[0] * (nl_exp / jnp.sum(nl_exp, axis=0, keepdims=True))
    logits = el + noise

    lmin = jnp.min(logits, axis=0, keepdims=True)
    eidx = jax.lax.broadcasted_iota(jnp.int32, logits.shape, 0)
    drop = jnp.max(jnp.where(logits == lmin, eidx, -1), axis=0, keepdims=True)
    keep = eidx != drop
    lmax = jnp.max(logits, axis=0, keepdims=True)
    ex = jnp.where(keep, jnp.exp(logits - lmax), 0.0)
    gates = ex / jnp.sum(ex, axis=0, keepdims=True)
    s = jnp.sum(gates, axis=0, keepdims=True)

    y = jnp.dot(
        wexp_ref[...].astype(jnp.bfloat16),
        xb16,
        preferred_element_type=jnp.float32,
    )
    o_ref[0] = (y + bexp_ref[...]) * s


def kernel(x, We, be, Wn, bn, Wexp, bexp, noise_uniform):
    B, C, N, P = x.shape
    E = We.shape[0]
    O = Wexp.shape[0]
    NP = N * P
    T = 2048

    x3 = x.reshape(B, C, NP)
    u3 = jnp.transpose(noise_uniform.reshape(B, NP, E), (0, 2, 1))
    bexp2 = bexp.reshape(O, 1)
    Wg = jnp.concatenate([We, Wn], axis=0)

    out = pl.pallas_call(
        _moe_block,
        grid=(B, NP // T),
        in_specs=[
            pl.BlockSpec((1, C, T), lambda b, i: (b, 0, i)),
            pl.BlockSpec((1, E, T), lambda b, i: (b, 0, i)),
            pl.BlockSpec((2 * E, C), lambda b, i: (0, 0)),
            pl.BlockSpec((O, C), lambda b, i: (0, 0)),
            pl.BlockSpec((O, 1), lambda b, i: (0, 0)),
        ],
        out_specs=pl.BlockSpec((1, O, T), lambda b, i: (b, 0, i)),
        out_shape=jax.ShapeDtypeStruct((B, O, NP), x.dtype),
    )(x3, u3, Wg, Wexp, bexp2)
    return out.reshape(B, O, N, P)
